# baseline (speedup 1.0000x reference)
import jax
import jax.numpy as jnp
from jax.experimental import pallas as pl
from jax.experimental.pallas import tpu as pltpu

A_DIM = 5
IMG_HW = 30
XS = 32
CTB = 64
KS1 = (0, 256, 512, 640)


def _band(w3, x_in, c_in, x_out, c_out):
    eyes = jnp.stack([jnp.eye(x_in, x_out, -kx, dtype=jnp.float32)
                      for kx in range(3)])
    acc = jnp.einsum('kio,kxz->xizo', w3, eyes)
    return acc.reshape(x_in * c_in, x_out * c_out)


def _fused_kernel(x_ref, wb0_ref, b0_ref, wb1_ref, b1_ref, wb2_ref, b2_ref,
                  wl1_ref, bl1_ref, wh_ref, bh_ref,
                  out_ref, a1_ref, a2_ref, f_ref):
    C = out_ref.shape[0]
    M0, M1, M2 = 28 * C, 26 * C, 24 * C

    xv = x_ref[...].reshape(M0, 96)
    acc = jnp.dot(xv, wb0_ref[...], preferred_element_type=jnp.float32)
    v0 = jnp.maximum(acc + b0_ref[...], 0.0).astype(jnp.bfloat16)
    for j in range(4):
        for ky in range(3):
            a1_ref[:, pl.ds(j * 1152 + ky * 384, 384)] = (
                v0[ky * C:ky * C + M1, KS1[j]:KS1[j] + 384])

    for j in range(4):
        acc = jnp.dot(a1_ref[:, pl.ds(j * 1152, 1152)], wb1_ref[j],
                      preferred_element_type=jnp.float32)
        vj = jnp.maximum(acc + b1_ref[:, pl.ds(256 * j, 256)],
                         0.0).astype(jnp.bfloat16)
        for ky in range(3):
            if j <= 2:
                a2_ref[:, pl.ds(j * 1152 + ky * 384, 256)] = (
                    vj[ky * C:ky * C + M2, :])
            if j >= 1:
                a2_ref[:, pl.ds((j - 1) * 1152 + ky * 384 + 256, 128)] = (
                    vj[ky * C:ky * C + M2, 0:128])

    for g in range(3):
        acc = jnp.dot(a2_ref[:, pl.ds(g * 1152, 1152)], wb2_ref[g],
                      preferred_element_type=jnp.float32)
        f_ref[:, pl.ds(128 * g, 128)] = jnp.maximum(
            acc + b2_ref[:, pl.ds(128 * g, 128)], 0.0).astype(jnp.bfloat16)

    h = jnp.dot(f_ref[pl.ds(0, C), :], wl1_ref[0],
                preferred_element_type=jnp.float32)
    for y in range(1, 24):
        h += jnp.dot(f_ref[pl.ds(y * C, C), :], wl1_ref[y],
                     preferred_element_type=jnp.float32)
    h = jnp.maximum(h + bl1_ref[...], 0.0).astype(jnp.bfloat16)

    out_ref[...] = jnp.dot(h, wh_ref[...],
                           preferred_element_type=jnp.float32) + bh_ref[...]


def kernel(img, w0, b0, w1, b1, w2, b2, wl1, bl1, wh, bh):
    B0 = img.shape[0]
    pad_b = (-B0) % CTB
    x = img.astype(jnp.float32)
    if pad_b:
        x = jnp.concatenate(
            [x, jnp.zeros((pad_b,) + x.shape[1:], jnp.float32)], axis=0)
    B = x.shape[0]

    x = jnp.transpose(x.reshape(B, IMG_HW, IMG_HW), (1, 0, 2))
    x = jnp.pad(x, ((0, 0), (0, 0), (0, XS - IMG_HW)))
    x3 = x.astype(jnp.bfloat16)
    x3c = jnp.concatenate([x3[0:28], x3[1:29], x3[2:30]], axis=2)

    wb0 = jnp.concatenate([_band(w0[3 * ky:3 * ky + 3].reshape(3, 1, 32),
                                 XS, 1, XS, 32) for ky in range(3)], axis=0)
    wb1f = [_band(w1[3 * ky:3 * ky + 3], XS, 32, XS, 32) for ky in range(3)]
    wb1 = jnp.stack([jnp.concatenate(
        [wb1f[ky][KS1[j]:KS1[j] + 384, 256 * j:256 * j + 256]
         for ky in range(3)], axis=0) for j in range(4)])
    wb2f = [_band(w2[3 * ky:3 * ky + 3], XS, 32, 24, 16) for ky in range(3)]
    wb2 = jnp.stack([jnp.concatenate(
        [wb2f[ky][256 * g:256 * g + 384, 128 * g:128 * g + 128]
         for ky in range(3)], axis=0) for g in range(3)])
    wb0 = wb0.astype(jnp.bfloat16)
    wb1 = wb1.astype(jnp.bfloat16)
    wb2 = wb2.astype(jnp.bfloat16)
    b0t = jnp.tile(b0, (1, XS))
    b1t = jnp.tile(b1, (1, XS))
    b2t = jnp.tile(b2, (1, 24))

    wl1r = wl1.reshape(24, 24 * 16, 128).astype(jnp.bfloat16)
    whb = wh.astype(jnp.bfloat16)

    heads = pl.pallas_call(
        _fused_kernel,
        out_shape=jax.ShapeDtypeStruct((B, 2 * A_DIM), jnp.float32),
        grid=(B // CTB,),
        in_specs=[
            pl.BlockSpec((28, CTB, 96), lambda i: (0, i, 0)),
            pl.BlockSpec((96, XS * 32), lambda i: (0, 0)),
            pl.BlockSpec((1, XS * 32), lambda i: (0, 0)),
            pl.BlockSpec((4, 1152, 256), lambda i: (0, 0, 0)),
            pl.BlockSpec((1, XS * 32), lambda i: (0, 0)),
            pl.BlockSpec((3, 1152, 128), lambda i: (0, 0, 0)),
            pl.BlockSpec((1, 24 * 16), lambda i: (0, 0)),
            pl.BlockSpec((24, 24 * 16, 128), lambda i: (0, 0, 0)),
            pl.BlockSpec((1, 128), lambda i: (0, 0)),
            pl.BlockSpec((128, 2 * A_DIM), lambda i: (0, 0)),
            pl.BlockSpec((1, 2 * A_DIM), lambda i: (0, 0)),
        ],
        out_specs=pl.BlockSpec((CTB, 2 * A_DIM), lambda i: (i, 0)),
        scratch_shapes=[pltpu.VMEM((26 * CTB, 4 * 1152), jnp.bfloat16),
                        pltpu.VMEM((24 * CTB, 3 * 1152), jnp.bfloat16),
                        pltpu.VMEM((24 * CTB, 24 * 16), jnp.bfloat16)],
        compiler_params=pltpu.CompilerParams(
            dimension_semantics=("parallel",)),
    )(x3c, wb0, b0t, wb1, b1t, wb2, b2t, wl1r, bl1, whb, bh)

    mean = heads[:B0, :A_DIM]
    log_std = heads[:B0, A_DIM:]
    return mean, log_std


# speedup vs baseline: 1.0111x; 1.0111x over previous
import jax
import jax.numpy as jnp
from jax.experimental import pallas as pl
from jax.experimental.pallas import tpu as pltpu

A_DIM = 5
IMG_HW = 30
XS = 32
CTB = 128
KS1 = (0, 256, 512, 640)


def _band(w3, x_in, c_in, x_out, c_out):
    eyes = jnp.stack([jnp.eye(x_in, x_out, -kx, dtype=jnp.float32)
                      for kx in range(3)])
    acc = jnp.einsum('kio,kxz->xizo', w3, eyes)
    return acc.reshape(x_in * c_in, x_out * c_out)


def _fused_kernel(x_ref, wb0_ref, b0_ref, wb1_ref, b1_ref, wb2_ref, b2_ref,
                  wl1_ref, bl1_ref, wh_ref, bh_ref,
                  out_ref, a1_ref, a2_ref, f_ref):
    C = out_ref.shape[0]
    M0, M1, M2 = 28 * C, 26 * C, 24 * C

    xv = x_ref[...].reshape(M0, 96)
    acc = jnp.dot(xv, wb0_ref[...], preferred_element_type=jnp.float32)
    a1_ref[...] = jnp.maximum(acc + b0_ref[...], 0.0).astype(jnp.bfloat16)

    for j in range(4):
        s = KS1[j]
        acc = jnp.dot(a1_ref[pl.ds(0, M1), pl.ds(s, 384)], wb1_ref[0, j],
                      preferred_element_type=jnp.float32)
        for ky in (1, 2):
            acc += jnp.dot(a1_ref[pl.ds(ky * C, M1), pl.ds(s, 384)],
                           wb1_ref[ky, j],
                           preferred_element_type=jnp.float32)
        a2_ref[:, pl.ds(256 * j, 256)] = jnp.maximum(
            acc + b1_ref[:, pl.ds(256 * j, 256)], 0.0).astype(jnp.bfloat16)

    for g in range(3):
        s = 256 * g
        acc = jnp.dot(a2_ref[pl.ds(0, M2), pl.ds(s, 384)], wb2_ref[0, g],
                      preferred_element_type=jnp.float32)
        for ky in (1, 2):
            acc += jnp.dot(a2_ref[pl.ds(ky * C, M2), pl.ds(s, 384)],
                           wb2_ref[ky, g],
                           preferred_element_type=jnp.float32)
        f_ref[:, pl.ds(128 * g, 128)] = jnp.maximum(
            acc + b2_ref[:, pl.ds(128 * g, 128)], 0.0).astype(jnp.bfloat16)

    h = jnp.dot(f_ref[pl.ds(0, C), :], wl1_ref[0],
                preferred_element_type=jnp.float32)
    for y in range(1, 24):
        h += jnp.dot(f_ref[pl.ds(y * C, C), :], wl1_ref[y],
                     preferred_element_type=jnp.float32)
    h = jnp.maximum(h + bl1_ref[...], 0.0).astype(jnp.bfloat16)

    out_ref[...] = jnp.dot(h, wh_ref[...],
                           preferred_element_type=jnp.float32) + bh_ref[...]


def kernel(img, w0, b0, w1, b1, w2, b2, wl1, bl1, wh, bh):
    B0 = img.shape[0]
    pad_b = (-B0) % CTB
    x = img
    if pad_b:
        x = jnp.concatenate(
            [x, jnp.zeros((pad_b,) + x.shape[1:], x.dtype)], axis=0)
    B = x.shape[0]

    xb = x.reshape(B, IMG_HW, IMG_HW).astype(jnp.bfloat16)
    xt = jnp.transpose(xb, (1, 0, 2))
    x3 = jnp.pad(xt, ((0, 0), (0, 0), (0, XS - IMG_HW)))
    x3c = jnp.concatenate([x3[0:28], x3[1:29], x3[2:30]], axis=2)

    wb0 = jnp.concatenate([_band(w0[3 * ky:3 * ky + 3].reshape(3, 1, 32),
                                 XS, 1, XS, 32) for ky in range(3)], axis=0)
    wb1f = [_band(w1[3 * ky:3 * ky + 3], XS, 32, XS, 32) for ky in range(3)]
    wb1 = jnp.stack([jnp.stack([wb1f[ky][KS1[j]:KS1[j] + 384,
                                         256 * j:256 * j + 256]
                                for j in range(4)]) for ky in range(3)])
    wb2f = [_band(w2[3 * ky:3 * ky + 3], XS, 32, 24, 16) for ky in range(3)]
    wb2 = jnp.stack([jnp.stack([wb2f[ky][256 * g:256 * g + 384,
                                         128 * g:128 * g + 128]
                                for g in range(3)]) for ky in range(3)])
    wb0 = wb0.astype(jnp.bfloat16)
    wb1 = wb1.astype(jnp.bfloat16)
    wb2 = wb2.astype(jnp.bfloat16)
    b0t = jnp.tile(b0, (1, XS))
    b1t = jnp.tile(b1, (1, XS))
    b2t = jnp.tile(b2, (1, 24))

    wl1r = wl1.reshape(24, 24 * 16, 128).astype(jnp.bfloat16)
    whb = wh.astype(jnp.bfloat16)

    heads = pl.pallas_call(
        _fused_kernel,
        out_shape=jax.ShapeDtypeStruct((B, 2 * A_DIM), jnp.float32),
        grid=(B // CTB,),
        in_specs=[
            pl.BlockSpec((28, CTB, 96), lambda i: (0, i, 0)),
            pl.BlockSpec((96, XS * 32), lambda i: (0, 0)),
            pl.BlockSpec((1, XS * 32), lambda i: (0, 0)),
            pl.BlockSpec((3, 4, 384, 256), lambda i: (0, 0, 0, 0)),
            pl.BlockSpec((1, XS * 32), lambda i: (0, 0)),
            pl.BlockSpec((3, 3, 384, 128), lambda i: (0, 0, 0, 0)),
            pl.BlockSpec((1, 24 * 16), lambda i: (0, 0)),
            pl.BlockSpec((24, 24 * 16, 128), lambda i: (0, 0, 0)),
            pl.BlockSpec((1, 128), lambda i: (0, 0)),
            pl.BlockSpec((128, 2 * A_DIM), lambda i: (0, 0)),
            pl.BlockSpec((1, 2 * A_DIM), lambda i: (0, 0)),
        ],
        out_specs=pl.BlockSpec((CTB, 2 * A_DIM), lambda i: (i, 0)),
        scratch_shapes=[pltpu.VMEM((28 * CTB, XS * 32), jnp.bfloat16),
                        pltpu.VMEM((26 * CTB, XS * 32), jnp.bfloat16),
                        pltpu.VMEM((24 * CTB, 24 * 16), jnp.bfloat16)],
        compiler_params=pltpu.CompilerParams(
            dimension_semantics=("parallel",)),
    )(x3c, wb0, b0t, wb1, b1t, wb2, b2t, wl1r, bl1, whb, bh)

    mean = heads[:B0, :A_DIM]
    log_std = heads[:B0, A_DIM:]
    return mean, log_std
